# baseline (speedup 1.0000x reference)
import math

import jax
import jax.numpy as jnp
from jax import lax
from jax.experimental import pallas as pl
from jax.experimental.pallas import tpu as pltpu
from jax.experimental.pallas import tpu_sc as plsc

D_MODEL = 64
SCALE = math.sqrt(D_MODEL)

NC = 2
NS = 16
NW = NC * NS

CH = 512
NSTREAM = 4
IDXW = CH // NSTREAM
L = 16


def _emb_body(idx_hbm, tab_hbm, out_hbm, idx_v, rows_v, sem):
    nch = idx_hbm.shape[0] // NW
    c = lax.axis_index("c")
    s = lax.axis_index("s")
    wid = s * NC + c
    blk0 = wid * nch

    def chunk(g, carry):
        blk = blk0 + g
        pltpu.sync_copy(idx_hbm.at[blk], idx_v)
        copies = []
        for j in range(NSTREAM):
            copies.append(pltpu.async_copy(
                tab_hbm.at[idx_v.at[j]],
                rows_v.at[pl.ds(j * IDXW, IDXW)],
                sem))
        for cp in copies:
            cp.wait()

        def mul_row(r, cc):
            for j in range(D_MODEL // L):
                v = rows_v[r, pl.ds(j * L, L)]
                rows_v[r, pl.ds(j * L, L)] = v * SCALE
            return cc

        lax.fori_loop(0, CH, mul_row, 0)
        pltpu.sync_copy(rows_v, out_hbm.at[blk])
        return carry

    lax.fori_loop(0, nch, chunk, 0)


@jax.jit
def _emb(xf, table):
    nblk = xf.shape[0]
    mesh = plsc.VectorSubcoreMesh(
        core_axis_name="c", subcore_axis_name="s",
        num_cores=NC, num_subcores=NS)
    f = pl.kernel(
        _emb_body,
        out_type=jax.ShapeDtypeStruct((nblk, CH, D_MODEL), jnp.float32),
        mesh=mesh,
        scratch_types=[
            pltpu.VMEM((NSTREAM, IDXW), jnp.int32),
            pltpu.VMEM((CH, D_MODEL), jnp.float32),
            pltpu.SemaphoreType.DMA,
        ],
        compiler_params=pltpu.CompilerParams(use_tc_tiling_on_sc=False),
    )
    return f(xf, table)


def kernel(x, table):
    b, t = x.shape
    n = b * t
    assert n % (NW * CH) == 0
    xf = x.reshape(n // CH, NSTREAM, IDXW).astype(jnp.int32)
    out = _emb(xf, table)
    return out.reshape(b, t, D_MODEL)
